# Initial kernel scaffold; baseline (speedup 1.0000x reference)
import jax
import jax.numpy as jnp
from jax.experimental import pallas as pl


def kernel(x, W1, b1, W2, b2):
    raise NotImplementedError("write your pallas kernel here")


# speedup vs baseline: 1.5091x; 1.5091x over previous
import jax
import jax.numpy as jnp
from jax import lax
from jax.experimental import pallas as pl

B, P, D, H = 512, 256, 128, 192
TB = 8


def _fused_kernel(x_ref, w1_ref, b1_ref, w2_ref, b2_ref, y_ref, m_ref, p_ref):
    x3 = x_ref[...]
    xb = x3.reshape(TB * P, D)

    h = jnp.dot(xb, w1_ref[...], preferred_element_type=jnp.float32)
    h = h + b1_ref[...]
    h = 0.5 * h * (1.0 + lax.erf(h * 0.7071067811865476))
    proj = jnp.dot(h, w2_ref[...], preferred_element_type=jnp.float32)
    proj = proj + b2_ref[...]

    nzc = jnp.sum((xb[:, :10] != 0.0).astype(jnp.float32), axis=1,
                  keepdims=True)
    y = jnp.where(nzc > 0.0, proj, 0.0)
    y_ref[...] = y.reshape(TB, P, H)

    nz3 = jnp.sum((x3[:, :, :10] != 0.0).astype(jnp.float32), axis=2)
    m_ref[...] = nz3 == 0.0

    p7 = x3[:, :, :7]
    idx = lax.broadcasted_iota(jnp.int32, (TB, P, 7), 2)
    p7 = jnp.where(idx < 4, p7, 0.0)
    p7 = jnp.where(idx == 5, 1.0, p7)
    p_ref[...] = p7


def kernel(x, W1, b1, W2, b2):
    b1r = b1.reshape(1, H)
    b2r = b2.reshape(1, H)
    grid = (B // TB,)
    y, mask, pos = pl.pallas_call(
        _fused_kernel,
        grid=grid,
        in_specs=[
            pl.BlockSpec((TB, P, D), lambda i: (i, 0, 0)),
            pl.BlockSpec((D, H), lambda i: (0, 0)),
            pl.BlockSpec((1, H), lambda i: (0, 0)),
            pl.BlockSpec((H, H), lambda i: (0, 0)),
            pl.BlockSpec((1, H), lambda i: (0, 0)),
        ],
        out_specs=[
            pl.BlockSpec((TB, P, H), lambda i: (i, 0, 0)),
            pl.BlockSpec((TB, P), lambda i: (i, 0)),
            pl.BlockSpec((TB, P, 7), lambda i: (i, 0, 0)),
        ],
        out_shape=[
            jax.ShapeDtypeStruct((B, P, H), jnp.float32),
            jax.ShapeDtypeStruct((B, P), jnp.bool_),
            jax.ShapeDtypeStruct((B, P, 7), jnp.float32),
        ],
    )(x, W1, b1r, W2, b2r)
    return (y, mask, pos)


# speedup vs baseline: 1.6463x; 1.0909x over previous
import jax
import jax.numpy as jnp
from jax import lax
from jax.experimental import pallas as pl

B, P, D, H = 512, 256, 128, 192
TB = 16


def _fused_kernel(x_ref, w1_ref, b1_ref, w2_ref, b2_ref, y_ref, m_ref, p_ref):
    x3 = x_ref[...]
    xb = x3.reshape(TB * P, D)

    h = jnp.dot(xb, w1_ref[...], preferred_element_type=jnp.float32)
    h = h + b1_ref[...]
    h = 0.5 * h * (1.0 + lax.erf(h * 0.7071067811865476))
    proj = jnp.dot(h, w2_ref[...], preferred_element_type=jnp.float32)
    proj = proj + b2_ref[...]

    nzc = jnp.sum((xb[:, :10] != 0.0).astype(jnp.float32), axis=1,
                  keepdims=True)
    y = jnp.where(nzc > 0.0, proj, 0.0)
    y_ref[...] = y.reshape(TB, P, H)

    nz3 = jnp.sum((x3[:, :, :10] != 0.0).astype(jnp.float32), axis=2)
    m_ref[...] = nz3 == 0.0

    p7 = x3[:, :, :7]
    idx = lax.broadcasted_iota(jnp.int32, (TB, P, 7), 2)
    p7 = jnp.where(idx < 4, p7, 0.0)
    p7 = jnp.where(idx == 5, 1.0, p7)
    p_ref[...] = p7


def kernel(x, W1, b1, W2, b2):
    b1r = b1.reshape(1, H)
    b2r = b2.reshape(1, H)
    grid = (B // TB,)
    y, mask, pos = pl.pallas_call(
        _fused_kernel,
        grid=grid,
        in_specs=[
            pl.BlockSpec((TB, P, D), lambda i: (i, 0, 0)),
            pl.BlockSpec((D, H), lambda i: (0, 0)),
            pl.BlockSpec((1, H), lambda i: (0, 0)),
            pl.BlockSpec((H, H), lambda i: (0, 0)),
            pl.BlockSpec((1, H), lambda i: (0, 0)),
        ],
        out_specs=[
            pl.BlockSpec((TB, P, H), lambda i: (i, 0, 0)),
            pl.BlockSpec((TB, P), lambda i: (i, 0)),
            pl.BlockSpec((TB, P, 7), lambda i: (i, 0, 0)),
        ],
        out_shape=[
            jax.ShapeDtypeStruct((B, P, H), jnp.float32),
            jax.ShapeDtypeStruct((B, P), jnp.bool_),
            jax.ShapeDtypeStruct((B, P, 7), jnp.float32),
        ],
    )(x, W1, b1r, W2, b2r)
    return (y, mask, pos)


# speedup vs baseline: 1.7115x; 1.0396x over previous
import jax
import jax.numpy as jnp
from jax import lax
from jax.experimental import pallas as pl

B, P, D, H = 512, 256, 128, 192
TB = 16


def _fused_kernel(x_ref, w1_ref, b1_ref, w2_ref, b2_ref, y_ref, m_ref):
    x3 = x_ref[...]
    xb = x3.reshape(TB * P, D)

    h = jnp.dot(xb, w1_ref[...], preferred_element_type=jnp.float32)
    h = h + b1_ref[...]
    h = 0.5 * h * (1.0 + lax.erf(h * 0.7071067811865476))
    proj = jnp.dot(h, w2_ref[...], preferred_element_type=jnp.float32)
    proj = proj + b2_ref[...]

    nzc = jnp.sum((xb[:, :10] != 0.0).astype(jnp.float32), axis=1,
                  keepdims=True)
    y = jnp.where(nzc > 0.0, proj, 0.0)
    y_ref[...] = y.reshape(TB, P, H)

    nz3 = jnp.sum((x3[:, :, :10] != 0.0).astype(jnp.float32), axis=2)
    m_ref[...] = nz3 == 0.0


def kernel(x, W1, b1, W2, b2):
    b1r = b1.reshape(1, H)
    b2r = b2.reshape(1, H)
    grid = (B // TB,)
    y, mask = pl.pallas_call(
        _fused_kernel,
        grid=grid,
        in_specs=[
            pl.BlockSpec((TB, P, D), lambda i: (i, 0, 0)),
            pl.BlockSpec((D, H), lambda i: (0, 0)),
            pl.BlockSpec((1, H), lambda i: (0, 0)),
            pl.BlockSpec((H, H), lambda i: (0, 0)),
            pl.BlockSpec((1, H), lambda i: (0, 0)),
        ],
        out_specs=[
            pl.BlockSpec((TB, P, H), lambda i: (i, 0, 0)),
            pl.BlockSpec((TB, P), lambda i: (i, 0)),
        ],
        out_shape=[
            jax.ShapeDtypeStruct((B, P, H), jnp.float32),
            jax.ShapeDtypeStruct((B, P), jnp.bool_),
        ],
    )(x, W1, b1r, W2, b2r)
    idx = lax.broadcasted_iota(jnp.int32, (B, P, 7), 2)
    p7 = x[:, :, :7]
    p7 = jnp.where(idx < 4, p7, 0.0)
    p7 = jnp.where(idx == 5, 1.0, p7)
    return (y, mask, p7)


# speedup vs baseline: 4.5378x; 2.6514x over previous
import jax
import jax.numpy as jnp
from jax import lax
from jax.experimental import pallas as pl

B, P, D, H = 512, 256, 128, 192
TB = 16


def _fused_kernel(x_ref, w1t_ref, b1_ref, w2t_ref, b2_ref, y_ref, m_ref, p_ref):
    R = TB * P
    xall = x_ref[...].reshape(R, D)
    xt = xall.T

    ht = jnp.dot(w1t_ref[...], xt, preferred_element_type=jnp.float32)
    ht = ht + b1_ref[...]
    ht = 0.5 * ht * (1.0 + lax.erf(ht * 0.7071067811865476))
    pt = jnp.dot(w2t_ref[...], ht, preferred_element_type=jnp.float32)
    pt = pt + b2_ref[...]

    nz = jnp.sum((xt[:10, :] != 0.0).astype(jnp.float32), axis=0,
                 keepdims=True)
    yt = jnp.where(nz > 0.0, pt, 0.0)

    p7 = xt[:7, :]
    idx = lax.broadcasted_iota(jnp.int32, (7, R), 0)
    p7 = jnp.where(idx < 4, p7, 0.0)
    p7 = jnp.where(idx == 5, 1.0, p7)

    for b in range(TB):
        lo, hi = b * P, (b + 1) * P
        y_ref[b] = yt[:, lo:hi]
        m_ref[b:b + 1, :] = nz[:, lo:hi] == 0.0
        p_ref[:, b, :] = p7[:, lo:hi]


def kernel(x, W1, b1, W2, b2):
    w1t = W1.T
    w2t = W2.T
    b1c = b1.reshape(H, 1)
    b2c = b2.reshape(H, 1)
    grid = (B // TB,)
    yt, mask, post = pl.pallas_call(
        _fused_kernel,
        grid=grid,
        in_specs=[
            pl.BlockSpec((TB, P, D), lambda i: (i, 0, 0)),
            pl.BlockSpec((H, D), lambda i: (0, 0)),
            pl.BlockSpec((H, 1), lambda i: (0, 0)),
            pl.BlockSpec((H, H), lambda i: (0, 0)),
            pl.BlockSpec((H, 1), lambda i: (0, 0)),
        ],
        out_specs=[
            pl.BlockSpec((TB, H, P), lambda i: (i, 0, 0)),
            pl.BlockSpec((TB, P), lambda i: (i, 0)),
            pl.BlockSpec((7, TB, P), lambda i: (0, i, 0)),
        ],
        out_shape=[
            jax.ShapeDtypeStruct((B, H, P), jnp.float32),
            jax.ShapeDtypeStruct((B, P), jnp.bool_),
            jax.ShapeDtypeStruct((7, B, P), jnp.float32),
        ],
    )(x, w1t, b1c, w2t, b2c)
    y = jnp.transpose(yt, (0, 2, 1))
    pos = jnp.transpose(post, (1, 2, 0))
    return (y, mask, pos)


# speedup vs baseline: 4.5412x; 1.0007x over previous
import jax
import jax.numpy as jnp
from jax import lax
from jax.experimental import pallas as pl
from jax.experimental.pallas import tpu as pltpu

B, P, D, H = 512, 256, 128, 192
TB = 16


def _fused_kernel(x_ref, w1t_ref, b1_ref, w2t_ref, b2_ref, y_ref, m_ref, p_ref):
    R = TB * P
    xall = x_ref[...].reshape(R, D)
    xt = xall.T

    ht = jnp.dot(w1t_ref[...], xt, preferred_element_type=jnp.float32)
    ht = ht + b1_ref[...]
    ht = 0.5 * ht * (1.0 + lax.erf(ht * 0.7071067811865476))
    pt = jnp.dot(w2t_ref[...], ht, preferred_element_type=jnp.float32)
    pt = pt + b2_ref[...]

    nz = jnp.sum((xt[:10, :] != 0.0).astype(jnp.float32), axis=0,
                 keepdims=True)
    yt = jnp.where(nz > 0.0, pt, 0.0)

    p7 = xt[:7, :]
    idx = lax.broadcasted_iota(jnp.int32, (7, R), 0)
    p7 = jnp.where(idx < 4, p7, 0.0)
    p7 = jnp.where(idx == 5, 1.0, p7)

    for b in range(TB):
        lo, hi = b * P, (b + 1) * P
        y_ref[b] = yt[:, lo:hi]
        m_ref[b:b + 1, :] = nz[:, lo:hi] == 0.0
        p_ref[:, b, :] = p7[:, lo:hi]


def kernel(x, W1, b1, W2, b2):
    w1t = W1.T
    w2t = W2.T
    b1c = b1.reshape(H, 1)
    b2c = b2.reshape(H, 1)
    grid = (B // TB,)
    yt, mask, post = pl.pallas_call(
        _fused_kernel,
        grid=grid,
        in_specs=[
            pl.BlockSpec((TB, P, D), lambda i: (i, 0, 0)),
            pl.BlockSpec((H, D), lambda i: (0, 0)),
            pl.BlockSpec((H, 1), lambda i: (0, 0)),
            pl.BlockSpec((H, H), lambda i: (0, 0)),
            pl.BlockSpec((H, 1), lambda i: (0, 0)),
        ],
        out_specs=[
            pl.BlockSpec((TB, H, P), lambda i: (i, 0, 0)),
            pl.BlockSpec((TB, P), lambda i: (i, 0)),
            pl.BlockSpec((7, TB, P), lambda i: (0, i, 0)),
        ],
        out_shape=[
            jax.ShapeDtypeStruct((B, H, P), jnp.float32),
            jax.ShapeDtypeStruct((B, P), jnp.bool_),
            jax.ShapeDtypeStruct((7, B, P), jnp.float32),
        ],
        compiler_params=pltpu.CompilerParams(
            dimension_semantics=("parallel",),
        ),
    )(x, w1t, b1c, w2t, b2c)
    y = jnp.transpose(yt, (0, 2, 1))
    pos = jnp.transpose(post, (1, 2, 0))
    return (y, mask, pos)


# speedup vs baseline: 5.0232x; 1.1061x over previous
import jax
import jax.numpy as jnp
from jax import lax
from jax.experimental import pallas as pl
from jax.experimental.pallas import tpu as pltpu

B, P, D, H = 512, 256, 128, 192
TB = 32


def _fused_kernel(x_ref, w1t_ref, b1_ref, w2t_ref, b2_ref, y_ref, m_ref, p_ref):
    R = TB * P
    xall = x_ref[...].reshape(R, D)
    xt = xall.T

    ht = jnp.dot(w1t_ref[...], xt, preferred_element_type=jnp.float32)
    ht = ht + b1_ref[...]
    ht = 0.5 * ht * (1.0 + lax.erf(ht * 0.7071067811865476))
    pt = jnp.dot(w2t_ref[...], ht, preferred_element_type=jnp.float32)
    pt = pt + b2_ref[...]

    nz = jnp.sum((xt[:10, :] != 0.0).astype(jnp.float32), axis=0,
                 keepdims=True)
    yt = jnp.where(nz > 0.0, pt, 0.0)

    p7 = xt[:7, :]
    idx = lax.broadcasted_iota(jnp.int32, (7, R), 0)
    p7 = jnp.where(idx < 4, p7, 0.0)
    p7 = jnp.where(idx == 5, 1.0, p7)

    for b in range(TB):
        lo, hi = b * P, (b + 1) * P
        y_ref[b] = yt[:, lo:hi]
        m_ref[b:b + 1, :] = nz[:, lo:hi] == 0.0
        p_ref[:, b, :] = p7[:, lo:hi]


def kernel(x, W1, b1, W2, b2):
    w1t = W1.T
    w2t = W2.T
    b1c = b1.reshape(H, 1)
    b2c = b2.reshape(H, 1)
    grid = (B // TB,)
    yt, mask, post = pl.pallas_call(
        _fused_kernel,
        grid=grid,
        in_specs=[
            pl.BlockSpec((TB, P, D), lambda i: (i, 0, 0)),
            pl.BlockSpec((H, D), lambda i: (0, 0)),
            pl.BlockSpec((H, 1), lambda i: (0, 0)),
            pl.BlockSpec((H, H), lambda i: (0, 0)),
            pl.BlockSpec((H, 1), lambda i: (0, 0)),
        ],
        out_specs=[
            pl.BlockSpec((TB, H, P), lambda i: (i, 0, 0)),
            pl.BlockSpec((TB, P), lambda i: (i, 0)),
            pl.BlockSpec((7, TB, P), lambda i: (0, i, 0)),
        ],
        out_shape=[
            jax.ShapeDtypeStruct((B, H, P), jnp.float32),
            jax.ShapeDtypeStruct((B, P), jnp.bool_),
            jax.ShapeDtypeStruct((7, B, P), jnp.float32),
        ],
        compiler_params=pltpu.CompilerParams(
            dimension_semantics=("parallel",),
        ),
    )(x, w1t, b1c, w2t, b2c)
    y = jnp.transpose(yt, (0, 2, 1))
    pos = jnp.transpose(post, (1, 2, 0))
    return (y, mask, pos)


# speedup vs baseline: 5.0335x; 1.0021x over previous
import jax
import jax.numpy as jnp
from jax import lax
from jax.experimental import pallas as pl
from jax.experimental.pallas import tpu as pltpu

B, P, D, H = 512, 256, 128, 192
TB = 64


def _fused_kernel(x_ref, w1t_ref, b1_ref, w2t_ref, b2_ref, y_ref, m_ref, p_ref):
    R = TB * P
    xall = x_ref[...].reshape(R, D)
    xt = xall.T

    ht = jnp.dot(w1t_ref[...], xt, preferred_element_type=jnp.float32)
    ht = ht + b1_ref[...]
    ht = 0.5 * ht * (1.0 + lax.erf(ht * 0.7071067811865476))
    pt = jnp.dot(w2t_ref[...], ht, preferred_element_type=jnp.float32)
    pt = pt + b2_ref[...]

    nz = jnp.sum((xt[:10, :] != 0.0).astype(jnp.float32), axis=0,
                 keepdims=True)
    yt = jnp.where(nz > 0.0, pt, 0.0)

    p7 = xt[:7, :]
    idx = lax.broadcasted_iota(jnp.int32, (7, R), 0)
    p7 = jnp.where(idx < 4, p7, 0.0)
    p7 = jnp.where(idx == 5, 1.0, p7)

    for b in range(TB):
        lo, hi = b * P, (b + 1) * P
        y_ref[b] = yt[:, lo:hi]
        m_ref[b:b + 1, :] = nz[:, lo:hi] == 0.0
        p_ref[:, b, :] = p7[:, lo:hi]


def kernel(x, W1, b1, W2, b2):
    w1t = W1.T
    w2t = W2.T
    b1c = b1.reshape(H, 1)
    b2c = b2.reshape(H, 1)
    grid = (B // TB,)
    yt, mask, post = pl.pallas_call(
        _fused_kernel,
        grid=grid,
        in_specs=[
            pl.BlockSpec((TB, P, D), lambda i: (i, 0, 0)),
            pl.BlockSpec((H, D), lambda i: (0, 0)),
            pl.BlockSpec((H, 1), lambda i: (0, 0)),
            pl.BlockSpec((H, H), lambda i: (0, 0)),
            pl.BlockSpec((H, 1), lambda i: (0, 0)),
        ],
        out_specs=[
            pl.BlockSpec((TB, H, P), lambda i: (i, 0, 0)),
            pl.BlockSpec((TB, P), lambda i: (i, 0)),
            pl.BlockSpec((7, TB, P), lambda i: (0, i, 0)),
        ],
        out_shape=[
            jax.ShapeDtypeStruct((B, H, P), jnp.float32),
            jax.ShapeDtypeStruct((B, P), jnp.bool_),
            jax.ShapeDtypeStruct((7, B, P), jnp.float32),
        ],
        compiler_params=pltpu.CompilerParams(
            dimension_semantics=("parallel",),
        ),
    )(x, w1t, b1c, w2t, b2c)
    y = jnp.transpose(yt, (0, 2, 1))
    pos = jnp.transpose(post, (1, 2, 0))
    return (y, mask, pos)


# speedup vs baseline: 5.1586x; 1.0249x over previous
import jax
import jax.numpy as jnp
from jax import lax
from jax.experimental import pallas as pl
from jax.experimental.pallas import tpu as pltpu

B, P, D, H = 512, 256, 128, 192
TB = 32


def _fused_kernel(x_ref, w1t_ref, b1_ref, w2t_ref, b2_ref, y_ref, m_ref, p_ref):
    R = TB * P
    xall = x_ref[...].reshape(R, D)
    xt = xall.T

    xtb = xt.astype(jnp.bfloat16)
    ht = lax.dot_general(w1t_ref[...], xtb, (((1,), (0,)), ((), ())),
                         preferred_element_type=jnp.float32)
    hb = (ht + b1_ref[...]).astype(jnp.bfloat16)
    one = jnp.bfloat16(1.0)
    half = jnp.bfloat16(0.5)
    isq2 = jnp.bfloat16(0.7071067811865476)
    g = half * hb * (one + lax.erf(hb * isq2))
    pt = lax.dot_general(w2t_ref[...], g, (((1,), (0,)), ((), ())),
                         preferred_element_type=jnp.float32)
    pt = pt + b2_ref[...]

    nz = jnp.sum((xt[:10, :] != 0.0).astype(jnp.float32), axis=0,
                 keepdims=True)
    yt = jnp.where(nz > 0.0, pt, 0.0)

    p7 = xt[:7, :]
    idx = lax.broadcasted_iota(jnp.int32, (7, R), 0)
    p7 = jnp.where(idx < 4, p7, 0.0)
    p7 = jnp.where(idx == 5, 1.0, p7)

    for b in range(TB):
        lo, hi = b * P, (b + 1) * P
        y_ref[b] = yt[:, lo:hi]
        m_ref[b:b + 1, :] = nz[:, lo:hi] == 0.0
        p_ref[:, b, :] = p7[:, lo:hi]


def kernel(x, W1, b1, W2, b2):
    w1t = W1.T.astype(jnp.bfloat16)
    w2t = W2.T.astype(jnp.bfloat16)
    b1c = b1.reshape(H, 1)
    b2c = b2.reshape(H, 1)
    grid = (B // TB,)
    yt, mask, post = pl.pallas_call(
        _fused_kernel,
        grid=grid,
        in_specs=[
            pl.BlockSpec((TB, P, D), lambda i: (i, 0, 0)),
            pl.BlockSpec((H, D), lambda i: (0, 0)),
            pl.BlockSpec((H, 1), lambda i: (0, 0)),
            pl.BlockSpec((H, H), lambda i: (0, 0)),
            pl.BlockSpec((H, 1), lambda i: (0, 0)),
        ],
        out_specs=[
            pl.BlockSpec((TB, H, P), lambda i: (i, 0, 0)),
            pl.BlockSpec((TB, P), lambda i: (i, 0)),
            pl.BlockSpec((7, TB, P), lambda i: (0, i, 0)),
        ],
        out_shape=[
            jax.ShapeDtypeStruct((B, H, P), jnp.float32),
            jax.ShapeDtypeStruct((B, P), jnp.bool_),
            jax.ShapeDtypeStruct((7, B, P), jnp.float32),
        ],
        compiler_params=pltpu.CompilerParams(
            dimension_semantics=("parallel",),
        ),
    )(x, w1t, b1c, w2t, b2c)
    y = jnp.transpose(yt, (0, 2, 1))
    pos = jnp.transpose(post, (1, 2, 0))
    return (y, mask, pos)
